# Initial kernel scaffold; baseline (speedup 1.0000x reference)
import jax
import jax.numpy as jnp
from jax.experimental import pallas as pl


def kernel(x, edge_index, W1, b1, gamma1, beta1, W2, b2, gamma2, beta2, W3, b3, gamma3, beta3):
    raise NotImplementedError("write your pallas kernel here")


# speedup vs baseline: 10.4730x; 10.4730x over previous
import functools

import jax
import jax.numpy as jnp
from jax import lax
from jax.experimental import pallas as pl
from jax.experimental.pallas import tpu as pltpu
from jax.experimental.pallas import tpu_sc as plsc

N = 10000
D = 128
E = 320000

NC = 2
NS = 16
NW = NC * NS
EDGES_PER_TILE = E // NW
CHUNK = 80
NCHUNK = EDGES_PER_TILE // CHUNK
ROWS_PER_TILE = 624
ROWS_REM = N - NS * ROWS_PER_TILE

_mesh = plsc.VectorSubcoreMesh(core_axis_name="c", subcore_axis_name="s")


@functools.partial(
    pl.kernel,
    mesh=_mesh,
    out_type=jax.ShapeDtypeStruct((NC, N), jnp.int32),
    scratch_types=[
        pltpu.VMEM((CHUNK,), jnp.int32),
        pltpu.VMEM((CHUNK,), jnp.int32),
        pltpu.VMEM_SHARED((N,), jnp.int32),
        pltpu.SemaphoreType.DMA,
    ],
)
def _sc_counts(dst_hbm, zeros_hbm, ones_hbm, out_hbm, dbuf, ones_v, csh, sem):
    c = lax.axis_index("c")
    s = lax.axis_index("s")
    wid = c * NS + s

    pltpu.sync_copy(ones_hbm, ones_v)

    @pl.when(s == 0)
    def _():
        pltpu.sync_copy(zeros_hbm, csh)

    plsc.subcore_barrier()

    base = wid * EDGES_PER_TILE

    def body(i, carry):
        off = base + i * CHUNK
        pltpu.sync_copy(dst_hbm.at[pl.ds(off, CHUNK)], dbuf)
        pltpu.sync_copy(ones_v, csh.at[dbuf], add=True)
        return carry

    lax.fori_loop(0, NCHUNK, body, 0)
    plsc.subcore_barrier()

    @pl.when(s == 0)
    def _():
        pltpu.sync_copy(csh, out_hbm.at[c])


@functools.partial(
    pl.kernel,
    mesh=_mesh,
    out_type=jax.ShapeDtypeStruct((NC, N, D), jnp.float32),
    scratch_types=[
        pltpu.VMEM((CHUNK,), jnp.int32),
        pltpu.VMEM((CHUNK, D), jnp.float32),
        pltpu.VMEM_SHARED((N, D), jnp.float32),
        pltpu.SemaphoreType.DMA,
    ],
)
def _sc_scatter(y_hbm, src_hbm, dst_hbm, zf_hbm, out_hbm, ibuf, rows, aggsh, sem):
    c = lax.axis_index("c")
    s = lax.axis_index("s")
    wid = c * NS + s

    pltpu.sync_copy(
        zf_hbm.at[pl.ds(s * ROWS_PER_TILE, ROWS_PER_TILE)],
        aggsh.at[pl.ds(s * ROWS_PER_TILE, ROWS_PER_TILE)],
    )

    @pl.when(s == 0)
    def _():
        pltpu.sync_copy(
            zf_hbm.at[pl.ds(NS * ROWS_PER_TILE, ROWS_REM)],
            aggsh.at[pl.ds(NS * ROWS_PER_TILE, ROWS_REM)],
        )

    plsc.subcore_barrier()

    base = wid * EDGES_PER_TILE

    def body(i, carry):
        off = base + i * CHUNK
        pltpu.sync_copy(src_hbm.at[pl.ds(off, CHUNK)], ibuf)
        pltpu.async_copy(y_hbm.at[ibuf], rows, sem).wait()
        pltpu.sync_copy(dst_hbm.at[pl.ds(off, CHUNK)], ibuf)
        pltpu.sync_copy(rows, aggsh.at[ibuf], add=True)
        return carry

    lax.fori_loop(0, NCHUNK, body, 0)
    plsc.subcore_barrier()

    pltpu.sync_copy(
        aggsh.at[pl.ds(s * ROWS_PER_TILE, ROWS_PER_TILE)],
        out_hbm.at[c, pl.ds(s * ROWS_PER_TILE, ROWS_PER_TILE)],
    )

    @pl.when(s == 0)
    def _():
        pltpu.sync_copy(
            aggsh.at[pl.ds(NS * ROWS_PER_TILE, ROWS_REM)],
            out_hbm.at[c, pl.ds(NS * ROWS_PER_TILE, ROWS_REM)],
        )


_BLK = 1000
_GRID = N // _BLK


def _pre_body(cnt_ref, x_ref, w_ref, dinv_ref, y_ref):
    cnt = cnt_ref[0] + cnt_ref[1] + 1
    dinv = lax.rsqrt(cnt.astype(jnp.float32))
    dinv_ref[...] = dinv
    y_ref[...] = jnp.dot(x_ref[...], w_ref[...],
                         preferred_element_type=jnp.float32) * dinv


_tc_pre = pl.pallas_call(
    _pre_body,
    grid=(_GRID,),
    in_specs=[
        pl.BlockSpec((NC, _BLK, 1), lambda i: (0, i, 0)),
        pl.BlockSpec((_BLK, D), lambda i: (i, 0)),
        pl.BlockSpec((D, D), lambda i: (0, 0)),
    ],
    out_specs=[
        pl.BlockSpec((_BLK, 1), lambda i: (i, 0)),
        pl.BlockSpec((_BLK, D), lambda i: (i, 0)),
    ],
    out_shape=[
        jax.ShapeDtypeStruct((N, 1), jnp.float32),
        jax.ShapeDtypeStruct((N, D), jnp.float32),
    ],
)


def _fuse_body(agg_ref, y_ref, dinv_ref, b_ref, h_ref, ps_ref, psq_ref,
               ps_acc, psq_acc):
    i = pl.program_id(0)
    a = agg_ref[0] + agg_ref[1] + y_ref[...]
    t = a * dinv_ref[...] + b_ref[...]
    h = jnp.maximum(t, 0.0)
    h_ref[...] = h
    s1 = jnp.sum(h, axis=0, keepdims=True)
    s2 = jnp.sum(h * h, axis=0, keepdims=True)

    @pl.when(i == 0)
    def _():
        ps_acc[...] = jnp.zeros_like(ps_acc)
        psq_acc[...] = jnp.zeros_like(psq_acc)

    ps_acc[...] += s1
    psq_acc[...] += s2

    @pl.when(i == _GRID - 1)
    def _():
        ps_ref[...] = ps_acc[...]
        psq_ref[...] = psq_acc[...]


_tc_fuse = pl.pallas_call(
    _fuse_body,
    grid=(_GRID,),
    in_specs=[
        pl.BlockSpec((NC, _BLK, D), lambda i: (0, i, 0)),
        pl.BlockSpec((_BLK, D), lambda i: (i, 0)),
        pl.BlockSpec((_BLK, 1), lambda i: (i, 0)),
        pl.BlockSpec((1, D), lambda i: (0, 0)),
    ],
    out_specs=[
        pl.BlockSpec((_BLK, D), lambda i: (i, 0)),
        pl.BlockSpec((1, D), lambda i: (0, 0)),
        pl.BlockSpec((1, D), lambda i: (0, 0)),
    ],
    out_shape=[
        jax.ShapeDtypeStruct((N, D), jnp.float32),
        jax.ShapeDtypeStruct((1, D), jnp.float32),
        jax.ShapeDtypeStruct((1, D), jnp.float32),
    ],
    scratch_shapes=[
        pltpu.VMEM((1, D), jnp.float32),
        pltpu.VMEM((1, D), jnp.float32),
    ],
)


def _bn_scale_shift(ps_ref, psq_ref, g_ref, be_ref):
    mean = ps_ref[0] / N
    ex2 = psq_ref[0] / N
    var = ex2 - mean * mean
    sc = g_ref[0] * lax.rsqrt(var + 1e-5)
    sh = be_ref[0] - mean * sc
    return sc, sh


def _next_body(h_ref, ps_ref, psq_ref, g_ref, be_ref, dinv_ref, w_ref, y_ref):
    sc, sh = _bn_scale_shift(ps_ref, psq_ref, g_ref, be_ref)
    z = h_ref[...] * sc[None, :] + sh[None, :]
    y_ref[...] = jnp.dot(z, w_ref[...],
                         preferred_element_type=jnp.float32) * dinv_ref[...]


_tc_next = pl.pallas_call(
    _next_body,
    grid=(_GRID,),
    in_specs=[
        pl.BlockSpec((_BLK, D), lambda i: (i, 0)),
        pl.BlockSpec((1, D), lambda i: (0, 0)),
        pl.BlockSpec((1, D), lambda i: (0, 0)),
        pl.BlockSpec((1, D), lambda i: (0, 0)),
        pl.BlockSpec((1, D), lambda i: (0, 0)),
        pl.BlockSpec((_BLK, 1), lambda i: (i, 0)),
        pl.BlockSpec((D, D), lambda i: (0, 0)),
    ],
    out_specs=pl.BlockSpec((_BLK, D), lambda i: (i, 0)),
    out_shape=jax.ShapeDtypeStruct((N, D), jnp.float32),
)


def _final_body(h_ref, ps_ref, psq_ref, g_ref, be_ref, out_ref):
    sc, sh = _bn_scale_shift(ps_ref, psq_ref, g_ref, be_ref)
    out_ref[...] = h_ref[...] * sc[None, :] + sh[None, :]


_tc_final = pl.pallas_call(
    _final_body,
    grid=(_GRID,),
    in_specs=[
        pl.BlockSpec((_BLK, D), lambda i: (i, 0)),
        pl.BlockSpec((1, D), lambda i: (0, 0)),
        pl.BlockSpec((1, D), lambda i: (0, 0)),
        pl.BlockSpec((1, D), lambda i: (0, 0)),
        pl.BlockSpec((1, D), lambda i: (0, 0)),
    ],
    out_specs=pl.BlockSpec((_BLK, D), lambda i: (i, 0)),
    out_shape=jax.ShapeDtypeStruct((N, D), jnp.float32),
)


def kernel(x, edge_index, W1, b1, gamma1, beta1, W2, b2, gamma2, beta2,
           W3, b3, gamma3, beta3):
    src = edge_index[0].astype(jnp.int32)
    dst = edge_index[1].astype(jnp.int32)

    zeros_i = jnp.zeros((N,), jnp.int32)
    zeros_f = jnp.zeros((N, D), jnp.float32)
    ones_i = jnp.ones((CHUNK,), jnp.int32)

    counts = _sc_counts(dst, zeros_i, ones_i)
    dinv, y = _tc_pre(counts.reshape(NC, N, 1), x, W1)

    params = [
        (b1, gamma1, beta1, W2),
        (b2, gamma2, beta2, W3),
        (b3, gamma3, beta3, None),
    ]
    out = None
    for b, g, be, w_next in params:
        aggs = _sc_scatter(y, src, dst, zeros_f)
        h, ps, psq = _tc_fuse(aggs, y, dinv, b.reshape(1, D))
        if w_next is not None:
            y = _tc_next(h, ps, psq, g.reshape(1, D), be.reshape(1, D),
                         dinv, w_next)
        else:
            out = _tc_final(h, ps, psq, g.reshape(1, D), be.reshape(1, D))
    return out
